# Initial kernel scaffold; baseline (speedup 1.0000x reference)
import jax
import jax.numpy as jnp
from jax.experimental import pallas as pl


def kernel(x_pep, x_ss_pep, x_2_pep, x_dense_pep, x_pretrain_pep, x_pro, x_ss_pro, x_2_pro, x_dense_pro, x_pretrain_pro, x_edge_pep, x_edge_pro, x_seqmask_pep, x_seqmask_pro, params):
    raise NotImplementedError("write your pallas kernel here")


# speedup vs baseline: 108.3866x; 108.3866x over previous
import functools

import jax
import jax.numpy as jnp
from jax.experimental import pallas as pl

_F32 = jnp.float32
_V_SEQ, _V_SS, _V_TWO = 25, 73, 8
_V_TOT = _V_SEQ + _V_SS + _V_TWO


def _body(idx_ref, xd_ref, xp_ref, adj_ref, mask_ref,
          emb_ref, encb_ref, wd_ref, wp_ref,
          w1_ref, b1_ref, w2_ref, b2_ref, wt_ref, bt_ref, out_ref):
    n = adj_ref.shape[-1]

    ids = idx_ref[0]
    k = jax.lax.broadcasted_iota(jnp.int32, (_V_TOT, n), 0)
    oh = ((k == ids[0:1, :])
          | (k == ids[1:2, :] + _V_SEQ)
          | (k == ids[2:3, :] + (_V_SEQ + _V_SS))).astype(_F32)
    emb_part = jnp.dot(emb_ref[...], oh, preferred_element_type=_F32)
    dense_part = jnp.dot(wd_ref[...], xd_ref[0], preferred_element_type=_F32)
    pre_part = jnp.dot(wp_ref[...], xp_ref[0], preferred_element_type=_F32)
    enc = jnp.concatenate([emb_part, dense_part, pre_part], axis=0) + encb_ref[...]
    mask = mask_ref[0]
    enc = enc * mask

    adj = adj_ref[0]
    deg = jnp.sum(adj, axis=0, keepdims=True) + 1.0
    dinv = jnp.where(deg > 0.0, jax.lax.rsqrt(deg), 0.0)

    def gcn(h, w_ref, b_ref):
        xw = jnp.dot(w_ref[...], h, preferred_element_type=_F32)
        y = xw * dinv
        agg = jnp.dot(y, adj, preferred_element_type=_F32) + y
        return agg * dinv + b_ref[...]

    h1 = jnp.maximum(gcn(enc, w1_ref, b1_ref), 0.0)
    h2 = gcn(h1, w2_ref, b2_ref)
    h = jnp.maximum(enc + h2, 0.0) * mask

    t = jnp.dot(wt_ref[...], h, preferred_element_type=_F32) + bt_ref[...]
    out_ref[0] = jnp.maximum(t, 0.0)


def _run(n, idx, xd_t, xp_t, adj, mask, emb_bd, enc_b, wd_t, wp_t,
         w1_t, b1, w2_t, b2, wt_t, bt):
    b = idx.shape[0]
    dd = xd_t.shape[1]
    batch = lambda *s: pl.BlockSpec((1,) + s, lambda i: (i, 0, 0))
    fixed = lambda *s: pl.BlockSpec(s, lambda i: (0, 0))
    return pl.pallas_call(
        _body,
        grid=(b,),
        in_specs=[
            batch(3, n),
            batch(dd, n),
            batch(1024, n),
            batch(n, n),
            batch(1, n),
            fixed(384, _V_TOT),
            fixed(640, 1),
            fixed(128, dd),
            fixed(128, 1024),
            fixed(640, 640),
            fixed(640, 1),
            fixed(640, 640),
            fixed(640, 1),
            fixed(896, 640),
            fixed(896, 1),
        ],
        out_specs=batch(896, n),
        out_shape=jax.ShapeDtypeStruct((b, 896, n), _F32),
    )(idx, xd_t, xp_t, adj, mask, emb_bd, enc_b, wd_t, wp_t,
      w1_t, b1, w2_t, b2, wt_t, bt)


def _prep_side(p, pfx, x_seq, x_ss, x_two, x_dense, x_pre, x_edge, x_mask):
    n = x_seq.shape[1]
    idx = jnp.stack([x_seq, x_ss, x_two], axis=1).astype(jnp.int32)
    xd_t = jnp.transpose(x_dense, (0, 2, 1))
    xp_t = jnp.transpose(x_pre, (0, 2, 1))
    mask = x_mask[:, None, :]

    z = functools.partial(jnp.zeros, dtype=_F32)
    emb_bd = jnp.concatenate([
        jnp.concatenate([p['embed_seq'].T, z((128, _V_SS + _V_TWO))], 1),
        jnp.concatenate([z((128, _V_SEQ)), p['embed_ss'].T, z((128, _V_TWO))], 1),
        jnp.concatenate([z((128, _V_SEQ + _V_SS)), p['embed_two'].T], 1),
    ], axis=0)
    enc_b = jnp.concatenate(
        [z((384,)), p['b_dense_' + pfx], p['b_pre_' + pfx]])[:, None]
    wt_t = jnp.transpose(p['W_' + pfx + '_trans'], (0, 2, 1)).reshape(896, 640)
    bt = p['b_' + pfx + '_trans'].reshape(896, 1)
    out = _run(n, idx, xd_t, xp_t, x_edge, mask, emb_bd, enc_b,
               p['W_dense_' + pfx].T, p['W_pre_' + pfx].T,
               p['W_gcn_' + pfx + '_1'].T, p['b_gcn_' + pfx + '_1'][:, None],
               p['W_gcn_' + pfx + '_2'].T, p['b_gcn_' + pfx + '_2'][:, None],
               wt_t, bt)
    return out


def kernel(x_pep, x_ss_pep, x_2_pep, x_dense_pep, x_pretrain_pep,
           x_pro, x_ss_pro, x_2_pro, x_dense_pro, x_pretrain_pro,
           x_edge_pep, x_edge_pro, x_seqmask_pep, x_seqmask_pro, params):
    p = params
    out_pep = _prep_side(p, 'pep', x_pep, x_ss_pep, x_2_pep,
                         x_dense_pep, x_pretrain_pep, x_edge_pep, x_seqmask_pep)
    out_pro = _prep_side(p, 'pro', x_pro, x_ss_pro, x_2_pro,
                         x_dense_pro, x_pretrain_pro, x_edge_pro, x_seqmask_pro)
    pep_vecs = tuple(out_pep[:, 128 * j:128 * (j + 1), :, None] for j in range(7))
    pro_vecs = tuple(out_pro[:, 128 * j:128 * (j + 1), None, :] for j in range(7))
    return (pep_vecs, pro_vecs)


# speedup vs baseline: 141.3178x; 1.3038x over previous
import functools

import jax
import jax.numpy as jnp
from jax.experimental import pallas as pl

_F32 = jnp.float32
_V_SEQ, _V_SS, _V_TWO = 25, 73, 8
_V_TOT = _V_SEQ + _V_SS + _V_TWO

_NHEAD = 7


def _dgT(a, b):
    return jax.lax.dot_general(a, b, (((0,), (0,)), ((), ())),
                               preferred_element_type=_F32)


def _dgTT(a, b):
    return jax.lax.dot_general(a, b, (((0,), (1,)), ((), ())),
                               preferred_element_type=_F32)


def _side(idx_ref, xd_ref, xp_ref, adj_ref, mask_ref, emb_ref, encb_ref,
          wd_ref, wp_ref, w1_ref, b1_ref, w2_ref, b2_ref, wt_ref, bt_ref,
          out_refs):
    n = adj_ref.shape[-1]

    ids = idx_ref[0]
    k = jax.lax.broadcasted_iota(jnp.int32, (_V_TOT, n), 0)
    oh = ((k == ids[0:1, :])
          | (k == ids[1:2, :] + _V_SEQ)
          | (k == ids[2:3, :] + (_V_SEQ + _V_SS))).astype(_F32)
    emb_part = jnp.dot(emb_ref[...], oh, preferred_element_type=_F32)
    dense_part = _dgTT(wd_ref[...], xd_ref[0])
    pre_part = _dgTT(wp_ref[...], xp_ref[0])
    enc = jnp.concatenate([emb_part, dense_part, pre_part], axis=0) + encb_ref[...]
    mask = mask_ref[0]
    enc = enc * mask

    adj = adj_ref[0]
    deg = jnp.sum(adj, axis=0, keepdims=True) + 1.0
    dinv = jnp.where(deg > 0.0, jax.lax.rsqrt(deg), 0.0)

    def gcn(h, w_ref, b_ref):
        xw = _dgT(w_ref[...], h)
        y = xw * dinv
        agg = jnp.dot(y, adj, preferred_element_type=_F32) + y
        return agg * dinv + b_ref[...]

    h1 = jnp.maximum(gcn(enc, w1_ref, b1_ref), 0.0)
    h2 = gcn(h1, w2_ref, b2_ref)
    h = jnp.maximum(enc + h2, 0.0) * mask

    for j in range(_NHEAD):
        t = _dgT(wt_ref[j], h) + bt_ref[j]
        out_refs[j][0] = jnp.maximum(t, 0.0)


def _body(*refs):
    a, b = refs[:15], refs[15:30]
    outs = refs[30:]
    _side(*a, outs[:_NHEAD])
    _side(*b, outs[_NHEAD:])


def _batch3(dd, n):
    return pl.BlockSpec((1, dd, n), lambda i: (i, 0, 0))


def _fixed(*s):
    return pl.BlockSpec(s, lambda i: tuple(0 for _ in s))


def _side_specs(p, pfx, n, dd, x_seq, x_ss, x_two, x_dense, x_pre, x_edge,
                x_mask):
    z = functools.partial(jnp.zeros, dtype=_F32)
    idx = jnp.stack([x_seq, x_ss, x_two], axis=1).astype(jnp.int32)
    emb_bd = jnp.concatenate([
        jnp.concatenate([p['embed_seq'].T, z((128, _V_SS + _V_TWO))], 1),
        jnp.concatenate([z((128, _V_SEQ)), p['embed_ss'].T, z((128, _V_TWO))], 1),
        jnp.concatenate([z((128, _V_SEQ + _V_SS)), p['embed_two'].T], 1),
    ], axis=0)
    enc_b = jnp.concatenate(
        [z((384,)), p['b_dense_' + pfx], p['b_pre_' + pfx]])[:, None]
    ops = [idx, x_dense, x_pre, x_edge, x_mask[:, None, :],
           emb_bd, enc_b, p['W_dense_' + pfx], p['W_pre_' + pfx],
           p['W_gcn_' + pfx + '_1'], p['b_gcn_' + pfx + '_1'][:, None],
           p['W_gcn_' + pfx + '_2'], p['b_gcn_' + pfx + '_2'][:, None],
           p['W_' + pfx + '_trans'], p['b_' + pfx + '_trans'][:, :, None]]
    specs = [_batch3(3, n), _batch3(n, dd), _batch3(n, 1024), _batch3(n, n),
             _batch3(1, n), _fixed(384, _V_TOT), _fixed(640, 1),
             _fixed(dd, 128), _fixed(1024, 128),
             _fixed(640, 640), _fixed(640, 1), _fixed(640, 640),
             _fixed(640, 1), _fixed(_NHEAD, 640, 128), _fixed(_NHEAD, 128, 1)]
    return ops, specs


def kernel(x_pep, x_ss_pep, x_2_pep, x_dense_pep, x_pretrain_pep,
           x_pro, x_ss_pro, x_2_pro, x_dense_pro, x_pretrain_pro,
           x_edge_pep, x_edge_pro, x_seqmask_pep, x_seqmask_pro, params):
    p = params
    bsz, lp = x_pep.shape
    lr = x_pro.shape[1]
    ops_p, specs_p = _side_specs(p, 'pep', lp, 3, x_pep, x_ss_pep, x_2_pep,
                                 x_dense_pep, x_pretrain_pep, x_edge_pep,
                                 x_seqmask_pep)
    ops_r, specs_r = _side_specs(p, 'pro', lr, 23, x_pro, x_ss_pro, x_2_pro,
                                 x_dense_pro, x_pretrain_pro, x_edge_pro,
                                 x_seqmask_pro)
    out_shapes = ([jax.ShapeDtypeStruct((bsz, 128, lp), _F32)] * _NHEAD
                  + [jax.ShapeDtypeStruct((bsz, 128, lr), _F32)] * _NHEAD)
    out_specs = ([_batch3(128, lp)] * _NHEAD + [_batch3(128, lr)] * _NHEAD)
    outs = pl.pallas_call(
        _body,
        grid=(bsz,),
        in_specs=specs_p + specs_r,
        out_specs=out_specs,
        out_shape=out_shapes,
    )(*ops_p, *ops_r)
    pep_vecs = tuple(o[:, :, :, None] for o in outs[:_NHEAD])
    pro_vecs = tuple(o[:, :, None, :] for o in outs[_NHEAD:])
    return (pep_vecs, pro_vecs)
